# baseline (speedup 1.0000x reference)
import jax
import jax.numpy as jnp
from jax.experimental import pallas as pl


def _vq_kernel(z_ref, cb_ref, z2_ref, cb2_ref, zq_ref, idx_ref, loss_ref):
    z = z_ref[...]
    cb = cb_ref[...]
    prod = jax.lax.dot_general(
        z.astype(jnp.bfloat16), cb.astype(jnp.bfloat16), (((1,), (1,)), ((), ())),
        preferred_element_type=jnp.float32)
    d = z2_ref[...] + cb2_ref[...] - 2.0 * prod
    dmin = jnp.min(d, axis=1, keepdims=True)
    iota = jax.lax.broadcasted_iota(jnp.int32, d.shape, 1)
    onehot_mask = d == dmin
    idx = jnp.min(jnp.where(onehot_mask, iota, 2**30), axis=1).astype(jnp.int32)
    onehot = (iota == idx[:, None]).astype(jnp.float32)
    zq = jax.lax.dot_general(
        onehot, cb, (((1,), (0,)), ((), ())),
        preferred_element_type=jnp.float32)
    zq_ref[...] = zq
    idx_ref[0, 0, :] = idx
    part = jnp.sum((zq - z) ** 2)[None, None]

    @pl.when(pl.program_id(0) == 0)
    def _():
        loss_ref[...] = jnp.zeros_like(loss_ref)

    loss_ref[...] += part


def kernel(z_e, codebook):
    B, C, H, W = z_e.shape
    K, D = codebook.shape
    N = B * H * W
    T = 4096
    nchunks = N // T
    z_flat = jnp.transpose(z_e, (0, 2, 3, 1)).reshape(N, C)
    z2 = z_e[:, 0] ** 2
    for c in range(1, C):
        z2 = z2 + z_e[:, c] ** 2
    z2 = z2.reshape(N, 1)
    cb2 = codebook[:, 0] ** 2
    for c in range(1, D):
        cb2 = cb2 + codebook[:, c] ** 2
    cb2 = cb2[None, :]
    zq_flat, idx, loss_sum = pl.pallas_call(
        _vq_kernel,
        grid=(nchunks,),
        in_specs=[
            pl.BlockSpec((T, C), lambda i: (i, 0)),
            pl.BlockSpec((K, D), lambda i: (0, 0)),
            pl.BlockSpec((T, 1), lambda i: (i, 0)),
            pl.BlockSpec((1, K), lambda i: (0, 0)),
        ],
        out_specs=[
            pl.BlockSpec((T, C), lambda i: (i, 0)),
            pl.BlockSpec((1, 1, T), lambda i: (i, 0, 0)),
            pl.BlockSpec((1, 1), lambda i: (0, 0)),
        ],
        out_shape=[
            jax.ShapeDtypeStruct((N, C), jnp.float32),
            jax.ShapeDtypeStruct((nchunks, 1, T), jnp.int32),
            jax.ShapeDtypeStruct((1, 1), jnp.float32),
        ],
    )(z_flat, codebook, z2, cb2)
    z_q = jnp.transpose(zq_flat.reshape(B, H, W, C), (0, 3, 1, 2))
    loss = loss_sum[0, 0] * (1.25 / (B * C * H * W))
    return (z_q, loss, idx.reshape(N))


# speedup vs baseline: 8.9835x; 8.9835x over previous
import jax
import jax.numpy as jnp
from jax.experimental import pallas as pl


def _vq_kernel(z_ref, cb_ref, cb2_ref, zq_ref, idx_ref, loss_ref):
    z = z_ref[0]
    cb = cb_ref[...]
    z2 = z[0:1, :] * z[0:1, :]
    for c in range(1, z.shape[0]):
        z2 = z2 + z[c:c+1, :] * z[c:c+1, :]
    prod = jax.lax.dot_general(
        cb.astype(jnp.bfloat16), z.astype(jnp.bfloat16), (((1,), (0,)), ((), ())),
        preferred_element_type=jnp.float32)
    d = z2 + cb2_ref[...] - 2.0 * prod
    dmin = jnp.min(d, axis=0, keepdims=True)
    iota = jax.lax.broadcasted_iota(jnp.int32, d.shape, 0)
    idx = jnp.min(jnp.where(d == dmin, iota, 2**30), axis=0).astype(jnp.int32)
    onehot = (iota == idx[None, :]).astype(jnp.float32)
    zq = jax.lax.dot_general(
        cb, onehot, (((0,), (0,)), ((), ())),
        preferred_element_type=jnp.float32)
    zq_ref[0] = zq
    idx_ref[0, 0, :] = idx
    part = jnp.sum((zq - z) ** 2)[None, None]

    @pl.when(pl.program_id(0) == 0)
    def _():
        loss_ref[...] = jnp.zeros_like(loss_ref)

    loss_ref[...] += part


def kernel(z_e, codebook):
    B, C, H, W = z_e.shape
    K, D = codebook.shape
    HW = H * W
    N = B * HW
    zr = z_e.reshape(B, C, HW)
    cb2 = codebook[:, 0] ** 2
    for c in range(1, D):
        cb2 = cb2 + codebook[:, c] ** 2
    cb2 = cb2[:, None]
    zq, idx, loss_sum = pl.pallas_call(
        _vq_kernel,
        grid=(B,),
        in_specs=[
            pl.BlockSpec((1, C, HW), lambda i: (i, 0, 0)),
            pl.BlockSpec((K, D), lambda i: (0, 0)),
            pl.BlockSpec((K, 1), lambda i: (0, 0)),
        ],
        out_specs=[
            pl.BlockSpec((1, C, HW), lambda i: (i, 0, 0)),
            pl.BlockSpec((1, 1, HW), lambda i: (i, 0, 0)),
            pl.BlockSpec((1, 1), lambda i: (0, 0)),
        ],
        out_shape=[
            jax.ShapeDtypeStruct((B, C, HW), jnp.float32),
            jax.ShapeDtypeStruct((B, 1, HW), jnp.int32),
            jax.ShapeDtypeStruct((1, 1), jnp.float32),
        ],
    )(zr, codebook, cb2)
    z_q = zq.reshape(B, C, H, W)
    loss = loss_sum[0, 0] * (1.25 / (B * C * H * W))
    return (z_q, loss, idx.reshape(N))


# speedup vs baseline: 9.2770x; 1.0327x over previous
import jax
import jax.numpy as jnp
from jax.experimental import pallas as pl
from jax.experimental.pallas import tpu as pltpu


def _vq_kernel(z_ref, cb_ref, cb2_ref, zq_ref, idx_ref, loss_ref):
    z = z_ref[0]
    cb = cb_ref[...]
    K = cb.shape[0]
    z2 = z[0:1, :] * z[0:1, :]
    for c in range(1, z.shape[0]):
        z2 = z2 + z[c:c+1, :] * z[c:c+1, :]
    cb_m2 = cb.astype(jnp.bfloat16) * -2.0
    prod_m2 = jax.lax.dot_general(
        cb_m2, z.astype(jnp.bfloat16), (((1,), (0,)), ((), ())),
        preferred_element_type=jnp.float32)
    d = (z2 + cb2_ref[...]) + prod_m2
    dmin = jnp.min(d, axis=0, keepdims=True)
    iota = jax.lax.broadcasted_iota(jnp.int32, d.shape, 0)
    idx = jnp.min(jnp.where(d == dmin, iota, K), axis=0).astype(jnp.int32)
    onehot = (iota == idx[None, :]).astype(jnp.bfloat16)
    zq = jax.lax.dot_general(
        cb, onehot, (((0,), (0,)), ((), ())),
        preferred_element_type=jnp.float32)
    zq_ref[0] = zq
    idx_ref[0, 0, :] = idx
    part = jnp.sum((zq - z) ** 2)[None, None]

    @pl.when(pl.program_id(0) == 0)
    def _():
        loss_ref[...] = jnp.zeros_like(loss_ref)

    loss_ref[...] += part


def kernel(z_e, codebook):
    B, C, H, W = z_e.shape
    K, D = codebook.shape
    HW = H * W
    N = B * HW
    zr = z_e.reshape(B, C, HW)
    cb2 = codebook[:, 0] ** 2
    for c in range(1, D):
        cb2 = cb2 + codebook[:, c] ** 2
    cb2 = cb2[:, None]
    zq, idx, loss_sum = pl.pallas_call(
        _vq_kernel,
        grid=(B,),
        in_specs=[
            pl.BlockSpec((1, C, HW), lambda i: (i, 0, 0)),
            pl.BlockSpec((K, D), lambda i: (0, 0)),
            pl.BlockSpec((K, 1), lambda i: (0, 0)),
        ],
        out_specs=[
            pl.BlockSpec((1, C, HW), lambda i: (i, 0, 0)),
            pl.BlockSpec((1, 1, HW), lambda i: (i, 0, 0)),
            pl.BlockSpec((1, 1), lambda i: (0, 0)),
        ],
        out_shape=[
            jax.ShapeDtypeStruct((B, C, HW), jnp.float32),
            jax.ShapeDtypeStruct((B, 1, HW), jnp.int32),
            jax.ShapeDtypeStruct((1, 1), jnp.float32),
        ],
    )(zr, codebook, cb2)
    z_q = zq.reshape(B, C, H, W)
    loss = loss_sum[0, 0] * (1.25 / (B * C * H * W))
    return (z_q, loss, idx.reshape(N))


# speedup vs baseline: 9.4204x; 1.0155x over previous
import jax
import jax.numpy as jnp
from jax.experimental import pallas as pl
from jax.experimental.pallas import tpu as pltpu


def _vq_kernel(z_ref, cb_ref, cb2_ref, zq_ref, idx_ref, loss_ref):
    G = z_ref.shape[0]
    HW = z_ref.shape[2]
    z = jnp.concatenate([z_ref[g] for g in range(G)], axis=1)
    cb = cb_ref[...]
    K = cb.shape[0]
    z2 = z[0:1, :] * z[0:1, :]
    for c in range(1, z.shape[0]):
        z2 = z2 + z[c:c+1, :] * z[c:c+1, :]
    cb_m2 = cb.astype(jnp.bfloat16) * -2.0
    prod_m2 = jax.lax.dot_general(
        cb_m2, z.astype(jnp.bfloat16), (((1,), (0,)), ((), ())),
        preferred_element_type=jnp.float32)
    d = (z2 + cb2_ref[...]) + prod_m2
    dmin = jnp.min(d, axis=0, keepdims=True)
    iota = jax.lax.broadcasted_iota(jnp.int32, d.shape, 0)
    idx = jnp.min(jnp.where(d == dmin, iota, K), axis=0).astype(jnp.int32)
    onehot = (iota == idx[None, :]).astype(jnp.bfloat16)
    zq = jax.lax.dot_general(
        cb, onehot, (((0,), (0,)), ((), ())),
        preferred_element_type=jnp.float32)
    for g in range(G):
        zq_ref[g] = zq[:, g * HW:(g + 1) * HW]
        idx_ref[g, 0, :] = idx[g * HW:(g + 1) * HW]
    part = jnp.sum((zq - z) ** 2)[None, None]

    @pl.when(pl.program_id(0) == 0)
    def _():
        loss_ref[...] = jnp.zeros_like(loss_ref)

    loss_ref[...] += part


def kernel(z_e, codebook):
    B, C, H, W = z_e.shape
    K, D = codebook.shape
    HW = H * W
    N = B * HW
    zr = z_e.reshape(B, C, HW)
    cb2 = codebook[:, 0] ** 2
    for c in range(1, D):
        cb2 = cb2 + codebook[:, c] ** 2
    cb2 = cb2[:, None]
    G = 2
    zq, idx, loss_sum = pl.pallas_call(
        _vq_kernel,
        grid=(B // G,),
        in_specs=[
            pl.BlockSpec((G, C, HW), lambda i: (i, 0, 0)),
            pl.BlockSpec((K, D), lambda i: (0, 0)),
            pl.BlockSpec((K, 1), lambda i: (0, 0)),
        ],
        out_specs=[
            pl.BlockSpec((G, C, HW), lambda i: (i, 0, 0)),
            pl.BlockSpec((G, 1, HW), lambda i: (i, 0, 0)),
            pl.BlockSpec((1, 1), lambda i: (0, 0)),
        ],
        out_shape=[
            jax.ShapeDtypeStruct((B, C, HW), jnp.float32),
            jax.ShapeDtypeStruct((B, 1, HW), jnp.int32),
            jax.ShapeDtypeStruct((1, 1), jnp.float32),
        ],
    )(zr, codebook, cb2)
    z_q = zq.reshape(B, C, H, W)
    loss = loss_sum[0, 0] * (1.25 / (B * C * H * W))
    return (z_q, loss, idx.reshape(N))
